# baseline (speedup 1.0000x reference)
import jax
import jax.numpy as jnp
from jax import lax
from jax.experimental import pallas as pl
from jax.experimental.pallas import tpu as pltpu
from jax.experimental.pallas import tpu_sc as plsc

B, L = 4096, 200
V, T, A = 100000, 26, 52
D = 64

NC, NS, LANES = 2, 16, 16
NW = NC * NS
N = B * L
PER_W = N // NW
C = 128
N_CHUNKS = PER_W // C


def _body(ids_hbm, tts_hbm, adms_hbm, code_hbm, type_hbm, adm_hbm, out_hbm,
          ids_v, tts_v, adms_v, rows0, rows1, type_tab, adm_tab,
          gsem0, gsem1, wsem0, wsem1):
    wid = lax.axis_index("s") * NC + lax.axis_index("c")
    rows = (rows0, rows1)
    gsem = (gsem0, gsem1)
    wsem = (wsem0, wsem1)

    pltpu.sync_copy(ids_hbm.at[wid], ids_v)
    pltpu.sync_copy(tts_hbm.at[wid], tts_v)
    pltpu.sync_copy(adms_hbm.at[wid], adms_v)
    pltpu.sync_copy(type_hbm, type_tab)
    pltpu.sync_copy(adm_hbm, adm_tab)

    def fire_gather(g, b):
        pltpu.async_copy(code_hbm.at[ids_v.at[g]], rows[b], gsem[b])

    def wait_gather(b):
        pltpu.make_async_copy(code_hbm.at[ids_v.at[0]], rows[b],
                              gsem[b]).wait()

    def fire_write(g, b):
        pltpu.async_copy(rows[b], out_hbm.at[wid, g], wsem[b])

    def wait_write(b):
        pltpu.make_async_copy(rows[b], out_hbm.at[wid, 0], wsem[b]).wait()

    iota16 = lax.iota(jnp.int32, LANES)

    fire_gather(0, 0)

    @pl.loop(0, N_CHUNKS, step=2)
    def _outer(g0):
        for b in (0, 1):
            g = g0 + b
            ob = 1 - b

            @pl.when(g >= 1)
            def _():
                wait_write(ob)

            @pl.when(g + 1 < N_CHUNKS)
            def _():
                fire_gather(g + 1, ob)

            wait_gather(b)

            @pl.loop(0, C // LANES)
            def _grp(g2):
                tok16 = g2 * LANES + iota16
                tt16 = tts_v[g, pl.ds(g2 * LANES, LANES)]
                adm16 = adms_v[g, pl.ds(g2 * LANES, LANES)]

                @pl.loop(0, D, unroll=4)
                def _d(d):
                    d16 = jnp.full((LANES,), d, jnp.int32)
                    cv = plsc.load_gather(rows[b], [tok16, d16])
                    tv = plsc.load_gather(type_tab, [tt16, d16])
                    av = plsc.load_gather(adm_tab, [adm16, d16])
                    plsc.store_scatter(rows[b], [tok16, d16], cv + tv + av)

            fire_write(g, b)

    wait_write((N_CHUNKS - 1) % 2)


@jax.jit
def kernel(input_ids, token_types, adm_index, code_embed, type_embed,
           adm_embed):
    ids3 = input_ids.reshape(NW, N_CHUNKS, C)
    tts3 = token_types.reshape(NW, N_CHUNKS, C)
    adms3 = adm_index.reshape(NW, N_CHUNKS, C)

    mesh = plsc.VectorSubcoreMesh(core_axis_name="c", subcore_axis_name="s")
    out = pl.kernel(
        _body,
        out_type=jax.ShapeDtypeStruct((NW, N_CHUNKS, C, D), jnp.float32),
        mesh=mesh,
        compiler_params=pltpu.CompilerParams(use_tc_tiling_on_sc=False,
                                             needs_layout_passes=False),
        scratch_types=[
            pltpu.VMEM((N_CHUNKS, C), jnp.int32),
            pltpu.VMEM((N_CHUNKS, C), jnp.int32),
            pltpu.VMEM((N_CHUNKS, C), jnp.int32),
            pltpu.VMEM((C, D), jnp.float32),
            pltpu.VMEM((C, D), jnp.float32),
            pltpu.VMEM((T, D), jnp.float32),
            pltpu.VMEM((A, D), jnp.float32),
            pltpu.SemaphoreType.DMA,
            pltpu.SemaphoreType.DMA,
            pltpu.SemaphoreType.DMA,
            pltpu.SemaphoreType.DMA,
        ],
    )(ids3, tts3, adms3, code_embed, type_embed, adm_embed)
    return out.reshape(B, L, D)


# speedup vs baseline: 3.2783x; 3.2783x over previous
import jax
import jax.numpy as jnp
from jax import lax
from jax.experimental import pallas as pl
from jax.experimental.pallas import tpu as pltpu
from jax.experimental.pallas import tpu_sc as plsc

B, L = 4096, 200
V, T, A = 100000, 26, 52
D = 64

NC, NS, LANES = 2, 16, 16
NW = NC * NS
N = B * L
PER_W = N // NW
C = 128
N_CHUNKS = PER_W // C


def _body(ids_hbm, tts_hbm, adms_hbm, code_hbm, type_hbm, adm_hbm, out_hbm,
          ids_v, tts_v, adms_v, rows0, rows1, type_tab, adm_tab,
          gsem0, gsem1, wsem0, wsem1):
    wid = lax.axis_index("s") * NC + lax.axis_index("c")
    rows = (rows0, rows1)
    gsem = (gsem0, gsem1)
    wsem = (wsem0, wsem1)

    pltpu.sync_copy(ids_hbm.at[wid], ids_v)
    pltpu.sync_copy(tts_hbm.at[wid], tts_v)
    pltpu.sync_copy(adms_hbm.at[wid], adms_v)
    pltpu.sync_copy(type_hbm, type_tab)
    pltpu.sync_copy(adm_hbm, adm_tab)

    def fire_gather(g, b):
        pltpu.async_copy(code_hbm.at[ids_v.at[g]], rows[b], gsem[b])

    def wait_gather(b):
        pltpu.make_async_copy(code_hbm.at[ids_v.at[0]], rows[b],
                              gsem[b]).wait()

    def fire_write(g, b):
        pltpu.async_copy(rows[b], out_hbm.at[wid, g], wsem[b])

    def wait_write(b):
        pltpu.make_async_copy(rows[b], out_hbm.at[wid, 0], wsem[b]).wait()

    iota16 = lax.iota(jnp.int32, LANES)

    fire_gather(0, 0)

    @pl.loop(0, N_CHUNKS, step=2)
    def _outer(g0):
        for b in (0, 1):
            g = g0 + b
            ob = 1 - b

            @pl.when(g >= 1)
            def _():
                wait_write(ob)

            @pl.when(g + 1 < N_CHUNKS)
            def _():
                fire_gather(g + 1, ob)

            wait_gather(b)

            @pl.loop(0, C // LANES)
            def _grp(g2):
                tok16 = g2 * LANES + iota16
                tt16 = tts_v[g, pl.ds(g2 * LANES, LANES)]
                adm16 = adms_v[g, pl.ds(g2 * LANES, LANES)]

                @pl.loop(0, LANES, unroll=2)
                def _d(d):
                    fvec = lax.bitwise_and(iota16 + d, LANES - 1)
                    for c in range(D // LANES):
                        col16 = fvec + c * LANES
                        cv = plsc.load_gather(rows[b], [tok16, col16])
                        tv = plsc.load_gather(type_tab, [tt16, col16])
                        av = plsc.load_gather(adm_tab, [adm16, col16])
                        plsc.store_scatter(rows[b], [tok16, col16],
                                           cv + tv + av)

            fire_write(g, b)

    wait_write((N_CHUNKS - 1) % 2)


@jax.jit
def kernel(input_ids, token_types, adm_index, code_embed, type_embed,
           adm_embed):
    ids3 = input_ids.reshape(NW, N_CHUNKS, C)
    tts3 = token_types.reshape(NW, N_CHUNKS, C)
    adms3 = adm_index.reshape(NW, N_CHUNKS, C)

    mesh = plsc.VectorSubcoreMesh(core_axis_name="c", subcore_axis_name="s")
    out = pl.kernel(
        _body,
        out_type=jax.ShapeDtypeStruct((NW, N_CHUNKS, C, D), jnp.float32),
        mesh=mesh,
        compiler_params=pltpu.CompilerParams(use_tc_tiling_on_sc=False,
                                             needs_layout_passes=False),
        scratch_types=[
            pltpu.VMEM((N_CHUNKS, C), jnp.int32),
            pltpu.VMEM((N_CHUNKS, C), jnp.int32),
            pltpu.VMEM((N_CHUNKS, C), jnp.int32),
            pltpu.VMEM((C, D), jnp.float32),
            pltpu.VMEM((C, D), jnp.float32),
            pltpu.VMEM((T, D), jnp.float32),
            pltpu.VMEM((A, D), jnp.float32),
            pltpu.SemaphoreType.DMA,
            pltpu.SemaphoreType.DMA,
            pltpu.SemaphoreType.DMA,
            pltpu.SemaphoreType.DMA,
        ],
    )(ids3, tts3, adms3, code_embed, type_embed, adm_embed)
    return out.reshape(B, L, D)


# speedup vs baseline: 3.3134x; 1.0107x over previous
import jax
import jax.numpy as jnp
from jax import lax
from jax.experimental import pallas as pl
from jax.experimental.pallas import tpu as pltpu
from jax.experimental.pallas import tpu_sc as plsc

B, L = 4096, 200
V, T, A = 100000, 26, 52
D = 64

NC, NS, LANES = 2, 16, 16
NW = NC * NS
N = B * L
PER_W = N // NW
C = 128
N_CHUNKS = PER_W // C


def _body(ids_hbm, tts_hbm, adms_hbm, code_hbm, type_hbm, adm_hbm, out_hbm,
          ids_v, tts_v, adms_v, rows0, rows1, type_tab, adm_tab,
          gsem0, gsem1, wsem0, wsem1):
    wid = lax.axis_index("s") * NC + lax.axis_index("c")
    rows = (rows0, rows1)
    gsem = (gsem0, gsem1)
    wsem = (wsem0, wsem1)

    pltpu.sync_copy(ids_hbm.at[wid], ids_v)
    pltpu.sync_copy(tts_hbm.at[wid], tts_v)
    pltpu.sync_copy(adms_hbm.at[wid], adms_v)
    pltpu.sync_copy(type_hbm, type_tab)
    pltpu.sync_copy(adm_hbm, adm_tab)

    def fire_gather(g, b):
        pltpu.async_copy(code_hbm.at[ids_v.at[g]], rows[b], gsem[b])

    def wait_gather(b):
        pltpu.make_async_copy(code_hbm.at[ids_v.at[0]], rows[b],
                              gsem[b]).wait()

    def fire_write(g, b):
        pltpu.async_copy(rows[b], out_hbm.at[wid, g], wsem[b])

    def wait_write(b):
        pltpu.make_async_copy(rows[b], out_hbm.at[wid, 0], wsem[b]).wait()

    iota16 = lax.iota(jnp.int32, LANES)

    fire_gather(0, 0)

    @pl.loop(0, N_CHUNKS, step=2)
    def _outer(g0):
        for b in (0, 1):
            g = g0 + b
            ob = 1 - b

            @pl.when(g >= 1)
            def _():
                wait_write(ob)

            @pl.when(g + 1 < N_CHUNKS)
            def _():
                fire_gather(g + 1, ob)

            wait_gather(b)

            @pl.loop(0, C // LANES)
            def _grp(g2):
                tok16 = g2 * LANES + iota16
                tt16 = tts_v[g, pl.ds(g2 * LANES, LANES)]
                adm16 = adms_v[g, pl.ds(g2 * LANES, LANES)]

                @pl.loop(0, LANES, unroll=4)
                def _d(d):
                    fvec = lax.bitwise_and(iota16 + d, LANES - 1)
                    for c in range(D // LANES):
                        col16 = fvec + c * LANES
                        cv = plsc.load_gather(rows[b], [tok16, col16])
                        tv = plsc.load_gather(type_tab, [tt16, col16])
                        av = plsc.load_gather(adm_tab, [adm16, col16])
                        plsc.store_scatter(rows[b], [tok16, col16],
                                           cv + tv + av)

            fire_write(g, b)

    wait_write((N_CHUNKS - 1) % 2)


@jax.jit
def kernel(input_ids, token_types, adm_index, code_embed, type_embed,
           adm_embed):
    ids3 = input_ids.reshape(NW, N_CHUNKS, C)
    tts3 = token_types.reshape(NW, N_CHUNKS, C)
    adms3 = adm_index.reshape(NW, N_CHUNKS, C)

    mesh = plsc.VectorSubcoreMesh(core_axis_name="c", subcore_axis_name="s")
    out = pl.kernel(
        _body,
        out_type=jax.ShapeDtypeStruct((NW, N_CHUNKS, C, D), jnp.float32),
        mesh=mesh,
        compiler_params=pltpu.CompilerParams(use_tc_tiling_on_sc=False,
                                             needs_layout_passes=False),
        scratch_types=[
            pltpu.VMEM((N_CHUNKS, C), jnp.int32),
            pltpu.VMEM((N_CHUNKS, C), jnp.int32),
            pltpu.VMEM((N_CHUNKS, C), jnp.int32),
            pltpu.VMEM((C, D), jnp.float32),
            pltpu.VMEM((C, D), jnp.float32),
            pltpu.VMEM((T, D), jnp.float32),
            pltpu.VMEM((A, D), jnp.float32),
            pltpu.SemaphoreType.DMA,
            pltpu.SemaphoreType.DMA,
            pltpu.SemaphoreType.DMA,
            pltpu.SemaphoreType.DMA,
        ],
    )(ids3, tts3, adms3, code_embed, type_embed, adm_embed)
    return out.reshape(B, L, D)


# speedup vs baseline: 4.7778x; 1.4420x over previous
import jax
import jax.numpy as jnp
from jax import lax
from jax.experimental import pallas as pl
from jax.experimental.pallas import tpu as pltpu
from jax.experimental.pallas import tpu_sc as plsc

B, L = 4096, 200
V, T, A = 100000, 26, 52
D = 64

NC, NS, LANES = 2, 16, 16
NW = NC * NS
N = B * L
PER_W = N // NW
C = 128
N_CHUNKS = PER_W // C


def _body(ids_hbm, tts_hbm, adms_hbm, code_hbm, type_hbm, adm_hbm, out_hbm,
          ids_v, tts_v, adms_v, rows0, rows1, outb0, outb1, type_tab, adm_tab,
          gsem0, gsem1, wsem0, wsem1):
    wid = lax.axis_index("s") * NC + lax.axis_index("c")
    rows = (rows0, rows1)
    outb = (outb0, outb1)
    gsem = (gsem0, gsem1)
    wsem = (wsem0, wsem1)

    pltpu.sync_copy(ids_hbm.at[wid], ids_v)
    pltpu.sync_copy(tts_hbm.at[wid], tts_v)
    pltpu.sync_copy(adms_hbm.at[wid], adms_v)
    pltpu.sync_copy(type_hbm, type_tab)
    pltpu.sync_copy(adm_hbm, adm_tab)

    def fire_gather(g, b):
        pltpu.async_copy(code_hbm.at[ids_v.at[g]], rows[b], gsem[b])

    def wait_gather(b):
        pltpu.make_async_copy(code_hbm.at[ids_v.at[0]], rows[b],
                              gsem[b]).wait()

    def fire_write(g, b):
        pltpu.async_copy(outb[b], out_hbm.at[wid, g], wsem[b])

    def wait_write(b):
        pltpu.make_async_copy(outb[b], out_hbm.at[wid, 0], wsem[b]).wait()

    iota16 = lax.iota(jnp.int32, LANES)

    fire_gather(0, 0)

    @pl.loop(0, N_CHUNKS, step=2)
    def _outer(g0):
        for b in (0, 1):
            g = g0 + b
            ob = 1 - b

            @pl.when(g >= 1)
            def _():
                wait_write(ob)

            @pl.when(g + 1 < N_CHUNKS)
            def _():
                fire_gather(g + 1, ob)

            wait_gather(b)

            @pl.loop(0, C // LANES)
            def _grp(g2):
                tok16 = g2 * LANES + iota16
                tt16 = tts_v[g, pl.ds(g2 * LANES, LANES)]
                adm16 = adms_v[g, pl.ds(g2 * LANES, LANES)]

                @plsc.parallel_loop(0, LANES, unroll=4)
                def _d(d):
                    fvec = lax.bitwise_and(iota16 + d, LANES - 1)
                    for c in range(D // LANES):
                        col16 = fvec + c * LANES
                        cv = plsc.load_gather(rows[b], [tok16, col16])
                        tv = plsc.load_gather(type_tab, [tt16, col16])
                        av = plsc.load_gather(adm_tab, [adm16, col16])
                        plsc.store_scatter(outb[b], [tok16, col16],
                                           cv + tv + av)

            fire_write(g, b)

    wait_write((N_CHUNKS - 1) % 2)


@jax.jit
def kernel(input_ids, token_types, adm_index, code_embed, type_embed,
           adm_embed):
    ids3 = input_ids.reshape(NW, N_CHUNKS, C)
    tts3 = token_types.reshape(NW, N_CHUNKS, C)
    adms3 = adm_index.reshape(NW, N_CHUNKS, C)

    mesh = plsc.VectorSubcoreMesh(core_axis_name="c", subcore_axis_name="s")
    out = pl.kernel(
        _body,
        out_type=jax.ShapeDtypeStruct((NW, N_CHUNKS, C, D), jnp.float32),
        mesh=mesh,
        compiler_params=pltpu.CompilerParams(use_tc_tiling_on_sc=False,
                                             needs_layout_passes=False),
        scratch_types=[
            pltpu.VMEM((N_CHUNKS, C), jnp.int32),
            pltpu.VMEM((N_CHUNKS, C), jnp.int32),
            pltpu.VMEM((N_CHUNKS, C), jnp.int32),
            pltpu.VMEM((C, D), jnp.float32),
            pltpu.VMEM((C, D), jnp.float32),
            pltpu.VMEM((C, D), jnp.float32),
            pltpu.VMEM((C, D), jnp.float32),
            pltpu.VMEM((T, D), jnp.float32),
            pltpu.VMEM((A, D), jnp.float32),
            pltpu.SemaphoreType.DMA,
            pltpu.SemaphoreType.DMA,
            pltpu.SemaphoreType.DMA,
            pltpu.SemaphoreType.DMA,
        ],
    )(ids3, tts3, adms3, code_embed, type_embed, adm_embed)
    return out.reshape(B, L, D)


# speedup vs baseline: 5.0277x; 1.0523x over previous
import jax
import jax.numpy as jnp
from jax import lax
from jax.experimental import pallas as pl
from jax.experimental.pallas import tpu as pltpu
from jax.experimental.pallas import tpu_sc as plsc

B, L = 4096, 200
V, T, A = 100000, 26, 52
D = 64
DP = 128

NC, NS, LANES = 2, 16, 16
NW = NC * NS
N = B * L
PER_W = N // NW
C = 128
N_CHUNKS = PER_W // C


def _body(ids_hbm, pk_hbm, code_hbm, type_hbm, adm_hbm, out_hbm,
          ids_v, pk_v, rows0, rows1, outb0, outb1, type_tab, adm_tab,
          gsem0, gsem1, wsem0, wsem1):
    wid = lax.axis_index("s") * NC + lax.axis_index("c")
    rows = (rows0, rows1)
    outb = (outb0, outb1)
    gsem = (gsem0, gsem1)
    wsem = (wsem0, wsem1)

    pltpu.sync_copy(ids_hbm.at[wid], ids_v)
    pltpu.sync_copy(pk_hbm.at[wid], pk_v)
    pltpu.sync_copy(type_hbm, type_tab)
    pltpu.sync_copy(adm_hbm, adm_tab)

    def fire_gather(g, b):
        pltpu.async_copy(code_hbm.at[ids_v.at[g]], rows[b], gsem[b])

    def wait_gather(b):
        pltpu.make_async_copy(code_hbm.at[ids_v.at[0]], rows[b],
                              gsem[b]).wait()

    def fire_write(g, b):
        pltpu.async_copy(outb[b], out_hbm.at[wid, g], wsem[b])

    def wait_write(b):
        pltpu.make_async_copy(outb[b], out_hbm.at[wid, 0], wsem[b]).wait()

    iota16 = lax.iota(jnp.int32, LANES)

    fire_gather(0, 0)

    @pl.loop(0, N_CHUNKS, step=2)
    def _outer(g0):
        for b in (0, 1):
            g = g0 + b
            ob = 1 - b

            @pl.when(g >= 1)
            def _():
                wait_write(ob)

            @pl.when(g + 1 < N_CHUNKS)
            def _():
                fire_gather(g + 1, ob)

            wait_gather(b)

            @pl.loop(0, C // LANES)
            def _grp(g2):
                tok16 = g2 * LANES + iota16
                pk16 = pk_v[g, pl.ds(g2 * LANES, LANES)]
                tt16 = lax.shift_right_logical(pk16, 6)
                adm16 = lax.bitwise_and(pk16, 63)

                @plsc.parallel_loop(0, LANES, unroll=4)
                def _d(d):
                    fvec = lax.bitwise_and(iota16 + d, LANES - 1)
                    for c in range(D // LANES):
                        col16 = fvec + c * LANES
                        cv = plsc.load_gather(rows[b], [tok16, col16])
                        tv = plsc.load_gather(type_tab, [tt16, col16])
                        av = plsc.load_gather(adm_tab, [adm16, col16])
                        plsc.store_scatter(outb[b], [tok16, col16],
                                           cv + tv + av)

            fire_write(g, b)

    wait_write((N_CHUNKS - 1) % 2)


@jax.jit
def kernel(input_ids, token_types, adm_index, code_embed, type_embed,
           adm_embed):
    ids3 = input_ids.reshape(NW, N_CHUNKS, C)
    packed3 = (token_types * 64 + adm_index).reshape(NW, N_CHUNKS, C)

    mesh = plsc.VectorSubcoreMesh(core_axis_name="c", subcore_axis_name="s")
    out = pl.kernel(
        _body,
        out_type=jax.ShapeDtypeStruct((NW, N_CHUNKS, C, DP), jnp.float32),
        mesh=mesh,
        compiler_params=pltpu.CompilerParams(use_tc_tiling_on_sc=False,
                                             needs_layout_passes=False),
        scratch_types=[
            pltpu.VMEM((N_CHUNKS, C), jnp.int32),
            pltpu.VMEM((N_CHUNKS, C), jnp.int32),
            pltpu.VMEM((C, D), jnp.float32),
            pltpu.VMEM((C, D), jnp.float32),
            pltpu.VMEM((C, DP), jnp.float32),
            pltpu.VMEM((C, DP), jnp.float32),
            pltpu.VMEM((T, D), jnp.float32),
            pltpu.VMEM((A, D), jnp.float32),
            pltpu.SemaphoreType.DMA,
            pltpu.SemaphoreType.DMA,
            pltpu.SemaphoreType.DMA,
            pltpu.SemaphoreType.DMA,
        ],
    )(ids3, packed3, code_embed, type_embed, adm_embed)
    return out.reshape(B, L, DP)[:, :, :D]


# speedup vs baseline: 7.2712x; 1.4462x over previous
import jax
import jax.numpy as jnp
from jax import lax
from jax.experimental import pallas as pl
from jax.experimental.pallas import tpu as pltpu
from jax.experimental.pallas import tpu_sc as plsc

B, L = 4096, 200
V, T, A = 100000, 26, 52
D = 64
DP = 128

NC, NS, LANES = 2, 16, 16
NW = NC * NS
N = B * L
PER_W = N // NW
C = 256
N_CHUNKS = PER_W // C


def _body(ids_hbm, pk_hbm, code_hbm, type_hbm, adm_hbm, out_hbm,
          ids_v, pk_v, rows0, rows1, outb0, outb1, type_tab, adm_tab,
          gsem0, gsem1, wsem0, wsem1):
    wid = lax.axis_index("s") * NC + lax.axis_index("c")
    rows = (rows0, rows1)
    outb = (outb0, outb1)
    gsem = (gsem0, gsem1)
    wsem = (wsem0, wsem1)

    pltpu.sync_copy(ids_hbm.at[wid], ids_v)
    pltpu.sync_copy(pk_hbm.at[wid], pk_v)
    pltpu.sync_copy(type_hbm, type_tab)
    pltpu.sync_copy(adm_hbm, adm_tab)

    def fire_gather(g, b):
        pltpu.async_copy(code_hbm.at[ids_v.at[g]], rows[b], gsem[b])

    def wait_gather(b):
        pltpu.make_async_copy(code_hbm.at[ids_v.at[0]], rows[b],
                              gsem[b]).wait()

    def fire_write(g, b):
        pltpu.async_copy(outb[b],
                         out_hbm.at[wid, g, :, pl.ds(0, D)], wsem[b])

    def wait_write(b):
        pltpu.make_async_copy(outb[b],
                              out_hbm.at[wid, 0, :, pl.ds(0, D)],
                              wsem[b]).wait()

    iota16 = lax.iota(jnp.int32, LANES)

    fire_gather(0, 0)

    @pl.loop(0, N_CHUNKS, step=2)
    def _outer(g0):
        for b in (0, 1):
            g = g0 + b
            ob = 1 - b

            @pl.when(g >= 1)
            def _():
                wait_write(ob)

            @pl.when(g + 1 < N_CHUNKS)
            def _():
                fire_gather(g + 1, ob)

            wait_gather(b)

            @pl.loop(0, C // LANES)
            def _grp(g2):
                tok16 = g2 * LANES + iota16
                pk16 = pk_v[g, pl.ds(g2 * LANES, LANES)]
                tt16 = lax.shift_right_logical(pk16, 6)
                adm16 = lax.bitwise_and(pk16, 63)

                @plsc.parallel_loop(0, LANES, unroll=4)
                def _d(d):
                    fvec = lax.bitwise_and(iota16 + d, LANES - 1)
                    for c in range(D // LANES):
                        col16 = fvec + c * LANES
                        cv = plsc.load_gather(rows[b], [tok16, col16])
                        tv = plsc.load_gather(type_tab, [tt16, col16])
                        av = plsc.load_gather(adm_tab, [adm16, col16])
                        plsc.store_scatter(outb[b], [tok16, col16],
                                           cv + tv + av)

            fire_write(g, b)

    wait_write((N_CHUNKS - 1) % 2)


@jax.jit
def kernel(input_ids, token_types, adm_index, code_embed, type_embed,
           adm_embed):
    ids3 = input_ids.reshape(NW, N_CHUNKS, C)
    packed3 = (token_types * 64 + adm_index).reshape(NW, N_CHUNKS, C)

    mesh = plsc.VectorSubcoreMesh(core_axis_name="c", subcore_axis_name="s")
    out = pl.kernel(
        _body,
        out_type=jax.ShapeDtypeStruct((NW, N_CHUNKS, C, DP), jnp.float32),
        mesh=mesh,
        compiler_params=pltpu.CompilerParams(use_tc_tiling_on_sc=False,
                                             needs_layout_passes=False),
        scratch_types=[
            pltpu.VMEM((N_CHUNKS, C), jnp.int32),
            pltpu.VMEM((N_CHUNKS, C), jnp.int32),
            pltpu.VMEM((C, D), jnp.float32),
            pltpu.VMEM((C, D), jnp.float32),
            pltpu.VMEM((C, D), jnp.float32),
            pltpu.VMEM((C, D), jnp.float32),
            pltpu.VMEM((T, D), jnp.float32),
            pltpu.VMEM((A, D), jnp.float32),
            pltpu.SemaphoreType.DMA,
            pltpu.SemaphoreType.DMA,
            pltpu.SemaphoreType.DMA,
            pltpu.SemaphoreType.DMA,
        ],
    )(ids3, packed3, code_embed, type_embed, adm_embed)
    return out.reshape(B, L, DP)[:, :, :D]


# speedup vs baseline: 7.2948x; 1.0032x over previous
import jax
import jax.numpy as jnp
from jax import lax
from jax.experimental import pallas as pl
from jax.experimental.pallas import tpu as pltpu
from jax.experimental.pallas import tpu_sc as plsc

B, L = 4096, 200
V, T, A = 100000, 26, 52
D = 64
DP = 128

NC, NS, LANES = 2, 16, 16
NW = NC * NS
N = B * L
PER_W = N // NW
C = 512
N_CHUNKS = PER_W // C


def _body(ids_hbm, pk_hbm, code_hbm, type_hbm, adm_hbm, out_hbm,
          ids_v, pk_v, rows0, rows1, type_tab, adm_tab,
          gsem0, gsem1, wsem0, wsem1):
    wid = lax.axis_index("s") * NC + lax.axis_index("c")
    rows = (rows0, rows1)
    gsem = (gsem0, gsem1)
    wsem = (wsem0, wsem1)

    pltpu.sync_copy(ids_hbm.at[wid], ids_v)
    pltpu.sync_copy(pk_hbm.at[wid], pk_v)
    pltpu.sync_copy(type_hbm, type_tab)
    pltpu.sync_copy(adm_hbm, adm_tab)

    def fire_gather(g, b):
        pltpu.async_copy(code_hbm.at[ids_v.at[g]], rows[b], gsem[b])

    def wait_gather(b):
        pltpu.make_async_copy(code_hbm.at[ids_v.at[0]], rows[b],
                              gsem[b]).wait()

    def fire_write(g, b):
        pltpu.async_copy(rows[b],
                         out_hbm.at[wid, g, :, pl.ds(0, D)], wsem[b])

    def wait_write(b):
        pltpu.make_async_copy(rows[b],
                              out_hbm.at[wid, 0, :, pl.ds(0, D)],
                              wsem[b]).wait()

    iota16 = lax.iota(jnp.int32, LANES)

    fire_gather(0, 0)

    @pl.loop(0, N_CHUNKS, step=2)
    def _outer(g0):
        for b in (0, 1):
            g = g0 + b
            ob = 1 - b

            @pl.when(g >= 1)
            def _():
                wait_write(ob)

            @pl.when(g + 1 < N_CHUNKS)
            def _():
                fire_gather(g + 1, ob)

            wait_gather(b)

            @pl.loop(0, C // LANES)
            def _grp(g2):
                tok16 = g2 * LANES + iota16
                pk16 = pk_v[g, pl.ds(g2 * LANES, LANES)]
                tt16 = lax.shift_right_logical(pk16, 6)
                adm16 = lax.bitwise_and(pk16, 63)

                @plsc.parallel_loop(0, LANES, unroll=4)
                def _d(d):
                    fvec = lax.bitwise_and(iota16 + d, LANES - 1)
                    for c in range(D // LANES):
                        col16 = fvec + c * LANES
                        cv = plsc.load_gather(rows[b], [tok16, col16])
                        tv = plsc.load_gather(type_tab, [tt16, col16])
                        av = plsc.load_gather(adm_tab, [adm16, col16])
                        plsc.store_scatter(rows[b], [tok16, col16],
                                           cv + tv + av)

            fire_write(g, b)

    wait_write((N_CHUNKS - 1) % 2)


@jax.jit
def kernel(input_ids, token_types, adm_index, code_embed, type_embed,
           adm_embed):
    ids3 = input_ids.reshape(NW, N_CHUNKS, C)
    packed3 = (token_types * 64 + adm_index).reshape(NW, N_CHUNKS, C)

    mesh = plsc.VectorSubcoreMesh(core_axis_name="c", subcore_axis_name="s")
    out = pl.kernel(
        _body,
        out_type=jax.ShapeDtypeStruct((NW, N_CHUNKS, C, DP), jnp.float32),
        mesh=mesh,
        compiler_params=pltpu.CompilerParams(use_tc_tiling_on_sc=False,
                                             needs_layout_passes=False),
        scratch_types=[
            pltpu.VMEM((N_CHUNKS, C), jnp.int32),
            pltpu.VMEM((N_CHUNKS, C), jnp.int32),
            pltpu.VMEM((C, D), jnp.float32),
            pltpu.VMEM((C, D), jnp.float32),
            pltpu.VMEM((T, D), jnp.float32),
            pltpu.VMEM((A, D), jnp.float32),
            pltpu.SemaphoreType.DMA,
            pltpu.SemaphoreType.DMA,
            pltpu.SemaphoreType.DMA,
            pltpu.SemaphoreType.DMA,
        ],
    )(ids3, packed3, code_embed, type_embed, adm_embed)
    return out.reshape(B, L, DP)[:, :, :D]


# speedup vs baseline: 8.2768x; 1.1346x over previous
import jax
import jax.numpy as jnp
from jax import lax
from jax.experimental import pallas as pl
from jax.experimental.pallas import tpu as pltpu
from jax.experimental.pallas import tpu_sc as plsc

B, L = 4096, 200
V, T, A = 100000, 26, 52
D = 64
DP = 128

NC, NS, LANES = 2, 16, 16
NW = NC * NS
N = B * L
PER_W = N // NW
C = 512
N_CHUNKS = PER_W // C


def _body(ids_hbm, pk_hbm, code_hbm, type_hbm, adm_hbm, out_hbm,
          ids_v, pk_v, rows0, rows1, type_tab, adm_tab,
          gsem0, gsem1, wsem0, wsem1):
    wid = lax.axis_index("s") * NC + lax.axis_index("c")
    rows = (rows0, rows1)
    gsem = (gsem0, gsem1)
    wsem = (wsem0, wsem1)

    pltpu.sync_copy(ids_hbm.at[wid], ids_v)
    pltpu.sync_copy(pk_hbm.at[wid], pk_v)
    pltpu.sync_copy(type_hbm, type_tab)
    pltpu.sync_copy(adm_hbm, adm_tab)

    def fire_gather(g, b):
        pltpu.async_copy(code_hbm.at[ids_v.at[g]], rows[b], gsem[b])

    def wait_gather(b):
        pltpu.make_async_copy(code_hbm.at[ids_v.at[0]], rows[b],
                              gsem[b]).wait()

    def fire_write(g, b):
        pltpu.async_copy(rows[b],
                         out_hbm.at[wid, g, :, pl.ds(0, D)], wsem[b])

    def wait_write(b):
        pltpu.make_async_copy(rows[b],
                              out_hbm.at[wid, 0, :, pl.ds(0, D)],
                              wsem[b]).wait()

    iota16 = lax.iota(jnp.int32, LANES)

    fire_gather(0, 0)

    @pl.loop(0, N_CHUNKS, step=2)
    def _outer(g0):
        for b in (0, 1):
            g = g0 + b
            ob = 1 - b

            @pl.when(g >= 1)
            def _():
                wait_write(ob)

            @pl.when(g + 1 < N_CHUNKS)
            def _():
                fire_gather(g + 1, ob)

            wait_gather(b)

            @pl.loop(0, C // LANES)
            def _grp(g2):
                tok16 = g2 * LANES + iota16
                pk16 = pk_v[g, pl.ds(g2 * LANES, LANES)]
                tt16 = lax.shift_right_logical(pk16, 6)
                adm16 = lax.bitwise_and(pk16, 63)

                @plsc.parallel_loop(0, LANES, unroll=4)
                def _d(d):
                    fvec = lax.bitwise_and(iota16 + d, LANES - 1)
                    for c in range(D // LANES):
                        col16 = fvec + c * LANES
                        tv = plsc.load_gather(type_tab, [tt16, col16])
                        av = plsc.load_gather(adm_tab, [adm16, col16])
                        plsc.addupdate_scatter(rows[b], [tok16, col16],
                                               tv + av)

            fire_write(g, b)

    wait_write((N_CHUNKS - 1) % 2)


@jax.jit
def kernel(input_ids, token_types, adm_index, code_embed, type_embed,
           adm_embed):
    ids3 = input_ids.reshape(NW, N_CHUNKS, C)
    packed3 = (token_types * 64 + adm_index).reshape(NW, N_CHUNKS, C)

    mesh = plsc.VectorSubcoreMesh(core_axis_name="c", subcore_axis_name="s")
    out = pl.kernel(
        _body,
        out_type=jax.ShapeDtypeStruct((NW, N_CHUNKS, C, DP), jnp.float32),
        mesh=mesh,
        compiler_params=pltpu.CompilerParams(use_tc_tiling_on_sc=False,
                                             needs_layout_passes=False),
        scratch_types=[
            pltpu.VMEM((N_CHUNKS, C), jnp.int32),
            pltpu.VMEM((N_CHUNKS, C), jnp.int32),
            pltpu.VMEM((C, D), jnp.float32),
            pltpu.VMEM((C, D), jnp.float32),
            pltpu.VMEM((T, D), jnp.float32),
            pltpu.VMEM((A, D), jnp.float32),
            pltpu.SemaphoreType.DMA,
            pltpu.SemaphoreType.DMA,
            pltpu.SemaphoreType.DMA,
            pltpu.SemaphoreType.DMA,
        ],
    )(ids3, packed3, code_embed, type_embed, adm_embed)
    return out.reshape(B, L, DP)[:, :, :D]
